# baseline (speedup 1.0000x reference)
import functools

import jax
import jax.numpy as jnp
from jax import lax
from jax.experimental import pallas as pl
from jax.experimental.pallas import tpu as pltpu
from jax.experimental.pallas import tpu_sc as plsc

E = 64
T = 64
DM = 768
DF = 3072


def _gelu(v):
    return v * 0.5 * (1.0 + lax.erf(v * (2.0 ** -0.5)))


def _logits_body(x_ref, gw_ref, out_ref):
    out_ref[...] = lax.dot_general(
        gw_ref[...], x_ref[...], (((1,), (1,)), ((), ())),
        preferred_element_type=jnp.float32)


def _router_logits(xf, gate_w):
    return pl.pallas_call(
        _logits_body,
        out_shape=jax.ShapeDtypeStruct((E, T), jnp.float32),
    )(xf, gate_w)


def _shared_body(x_ref, ws1_ref, bs1_ref, ws2_ref, bs2_ref, out_ref):
    h = _gelu(
        lax.dot_general(x_ref[...], ws1_ref[...], (((1,), (1,)), ((), ())),
                        preferred_element_type=jnp.float32) + bs1_ref[...])
    out_ref[...] = lax.dot_general(
        h, ws2_ref[...], (((1,), (1,)), ((), ())),
        preferred_element_type=jnp.float32) + bs2_ref[...]


def _shared_expert(xf, Ws1, bs1r, Ws2, bs2r):
    return pl.pallas_call(
        _shared_body,
        out_shape=jax.ShapeDtypeStruct((T, DM), jnp.float32),
    )(xf, Ws1, bs1r, Ws2, bs2r)


def _dispatch(logits_t):
    def _lane_gather(v, idx):
        return lax.gather(
            v, idx[:, None],
            dimension_numbers=lax.GatherDimensionNumbers(
                offset_dims=(), collapsed_slice_dims=(0,),
                start_index_map=(0,)),
            slice_sizes=(1,),
            mode=lax.GatherScatterMode.PROMISE_IN_BOUNDS)

    mesh = plsc.VectorSubcoreMesh(core_axis_name="c", subcore_axis_name="s")

    @functools.partial(
        pl.kernel,
        mesh=mesh,
        out_type=[
            jax.ShapeDtypeStruct((T,), jnp.int32),
            jax.ShapeDtypeStruct((E,), jnp.int32),
        ],
        scratch_types=[
            pltpu.VMEM((E, T), jnp.float32),
            pltpu.VMEM((T,), jnp.int32),
            pltpu.VMEM((E,), jnp.int32),
        ],
    )
    def k(lg_hbm, sel_hbm, prev_hbm, lg_v, sel_v, prev_v):
        cid = lax.axis_index("c")
        sid = lax.axis_index("s")

        @pl.when(jnp.logical_and(cid == 0, sid == 0))
        def _():
            pltpu.sync_copy(lg_hbm, lg_v)
            zeros_i = jnp.zeros((16,), jnp.int32)
            neg = jnp.full((16,), -3.0e38, jnp.float32)
            iota = lax.iota(jnp.int32, 16)

            def body(e, carry):
                bv = list(carry[0:4])
                bi = list(carry[4:8])
                ev = jnp.full((16,), e, jnp.int32)
                for c in range(4):
                    v = lg_v[e, pl.ds(c * 16, 16)]
                    m = v > bv[c]
                    bv[c] = jnp.where(m, v, bv[c])
                    bi[c] = jnp.where(m, ev, bi[c])
                return tuple(bv) + tuple(bi)

            init = (neg, neg, neg, neg, zeros_i, zeros_i, zeros_i, zeros_i)
            res = lax.fori_loop(0, E, body, init)
            bi = res[4:8]
            for c in range(4):
                sel_v[pl.ds(c * 16, 16)] = bi[c]

            one = jnp.full((16,), 1, jnp.int32)
            zero = zeros_i
            lo = zero
            hi = zero
            for c in range(4):
                sh = jnp.bitwise_and(bi[c], 31)
                bit = jnp.left_shift(one, sh)
                lo = jnp.bitwise_or(lo, jnp.where(bi[c] < 32, bit, zero))
                hi = jnp.bitwise_or(hi, jnp.where(bi[c] >= 32, bit, zero))
            for off in (1, 2, 4, 8):
                ridx = jnp.bitwise_and(iota + off, 15)
                lo = jnp.bitwise_or(lo, _lane_gather(lo, ridx))
                hi = jnp.bitwise_or(hi, _lane_gather(hi, ridx))

            def fold(v, op):
                for off in (1, 2, 4, 8):
                    v = op(v, _lane_gather(v, jnp.bitwise_and(iota + off, 15)))
                return v

            def prefix_max(v):
                neg1 = jnp.full((16,), -1, jnp.int32)
                for off in (1, 2, 4, 8):
                    g = _lane_gather(v, jnp.maximum(iota - off, 0))
                    v = jnp.maximum(v, jnp.where(iota >= off, g, neg1))
                return v

            ev_big = jnp.full((16,), E, jnp.int32)
            f = ev_big
            us = []
            for ec in range(4):
                word = lo if ec < 2 else hi
                sh = iota + (16 if (ec % 2) else 0)
                u = jnp.bitwise_and(lax.shift_right_logical(word, sh), one)
                us.append(u)
                ids = iota + ec * 16
                f = jnp.minimum(f, fold(jnp.where(u == 1, ids, ev_big),
                                        jnp.minimum))

            carry = jnp.full((16,), -1, jnp.int32)
            for ec in range(4):
                ids = iota + ec * 16
                pc = prefix_max(jnp.where(us[ec] == 1, ids, -1))
                pc = jnp.maximum(pc, carry)
                carry = fold(pc, jnp.maximum)
                pc = jnp.where(pc < 0, f, pc)
                prev_v[pl.ds(ec * 16, 16)] = pc

            pltpu.sync_copy(sel_v, sel_hbm)
            pltpu.sync_copy(prev_v, prev_hbm)

    return k(logits_t)


NQ1 = 4
DQ1 = DF // NQ1
NQ2 = 6
DQ2 = DM // NQ2


def _moe_body(prev_sm, x_ref, *rest):
    w1_refs = rest[0:NQ1]
    w2_refs = rest[NQ1:NQ1 + NQ2]
    b1_ref, b2_ref, sh_ref, selv_ref, out_ref = rest[NQ1 + NQ2:]
    i = pl.program_id(0)

    @pl.when(i == 0)
    def _():
        out_ref[...] = sh_ref[...]

    @pl.when(prev_sm[i] == i)
    def _():
        hs = []
        for q in range(NQ1):
            xw = lax.dot_general(x_ref[...], w1_refs[q][0],
                                 (((1,), (1,)), ((), ())),
                                 preferred_element_type=jnp.float32)
            hs.append(_gelu(xw + b1_ref[0, :, pl.ds(q * DQ1, DQ1)]))
        h = jnp.concatenate(hs, axis=1)
        os = []
        for m in range(NQ2):
            os.append(lax.dot_general(h, w2_refs[m][0],
                                      (((1,), (1,)), ((), ())),
                                      preferred_element_type=jnp.float32))
        o = jnp.concatenate(os, axis=1) + b2_ref[0]
        mask = selv_ref[...] == i
        out_ref[...] += jnp.where(mask, o, 0.0)


def _moe(xf, W1, b1r, W2, b2r, shared, selv, prev):
    def w1q(q):
        return pl.BlockSpec((1, DQ1, DM), lambda i, p, q=q: (p[i], q, 0))

    def w2q(m):
        return pl.BlockSpec((1, DQ2, DF), lambda i, p, m=m: (p[i], m, 0))

    grid_spec = pltpu.PrefetchScalarGridSpec(
        num_scalar_prefetch=1,
        grid=(E,),
        in_specs=[
            pl.BlockSpec((T, DM), lambda i, *_: (0, 0)),
            *[w1q(q) for q in range(NQ1)],
            *[w2q(m) for m in range(NQ2)],
            pl.BlockSpec((1, 1, DF), lambda i, p: (p[i], 0, 0)),
            pl.BlockSpec((1, 1, DM), lambda i, p: (p[i], 0, 0)),
            pl.BlockSpec((T, DM), lambda i, *_: (0, 0)),
            pl.BlockSpec((T, 1), lambda i, *_: (0, 0)),
        ],
        out_specs=pl.BlockSpec((T, DM), lambda i, *_: (0, 0)),
    )
    return pl.pallas_call(
        _moe_body,
        grid_spec=grid_spec,
        out_shape=jax.ShapeDtypeStruct((T, DM), jnp.float32),
        compiler_params=pltpu.CompilerParams(
            dimension_semantics=("arbitrary",)),
    )(prev, xf, *[W1] * NQ1, *[W2] * NQ2, b1r, b2r, shared, selv)


def kernel(x, gate_w, W1, b1, W2, b2, Ws1, bs1, Ws2, bs2):
    bsz, seq, dm = x.shape
    xf = x.reshape(-1, dm)

    logits_t = _router_logits(xf, gate_w)
    shared = _shared_expert(xf, Ws1, bs1.reshape(1, DF), Ws2, bs2.reshape(1, DM))

    sel, prev = _dispatch(logits_t)
    selv = sel.reshape(T, 1)

    b1r = b1.reshape(E, 1, DF)
    b2r = b2.reshape(E, 1, DM)

    out = _moe(xf, W1, b1r, W2, b2r, shared, selv, prev)
    return out.reshape(bsz, seq, dm)


# speedup vs baseline: 1.2884x; 1.2884x over previous
import functools

import jax
import jax.numpy as jnp
from jax import lax
from jax.experimental import pallas as pl
from jax.experimental.pallas import tpu as pltpu
from jax.experimental.pallas import tpu_sc as plsc

E = 64
T = 64
DM = 768
DF = 3072


def _gelu(v):
    return v * 0.5 * (1.0 + lax.erf(v * (2.0 ** -0.5)))


def _logits_body(x_ref, gw_ref, out_ref):
    out_ref[...] = lax.dot_general(
        gw_ref[...], x_ref[...], (((1,), (1,)), ((), ())),
        preferred_element_type=jnp.float32)


def _router_logits(xf, gate_w):
    return pl.pallas_call(
        _logits_body,
        out_shape=jax.ShapeDtypeStruct((E, T), jnp.float32),
    )(xf, gate_w)


def _shared_body(x_ref, ws1_ref, bs1_ref, ws2_ref, bs2_ref, out_ref):
    h = _gelu(
        lax.dot_general(x_ref[...], ws1_ref[...], (((1,), (1,)), ((), ())),
                        preferred_element_type=jnp.float32) + bs1_ref[...])
    out_ref[...] = lax.dot_general(
        h, ws2_ref[...], (((1,), (1,)), ((), ())),
        preferred_element_type=jnp.float32) + bs2_ref[...]


def _shared_expert(xf, Ws1, bs1r, Ws2, bs2r):
    return pl.pallas_call(
        _shared_body,
        out_shape=jax.ShapeDtypeStruct((T, DM), jnp.float32),
    )(xf, Ws1, bs1r, Ws2, bs2r)


def _dispatch(logits_t):
    def _lane_gather(v, idx):
        return lax.gather(
            v, idx[:, None],
            dimension_numbers=lax.GatherDimensionNumbers(
                offset_dims=(), collapsed_slice_dims=(0,),
                start_index_map=(0,)),
            slice_sizes=(1,),
            mode=lax.GatherScatterMode.PROMISE_IN_BOUNDS)

    mesh = plsc.VectorSubcoreMesh(core_axis_name="c", subcore_axis_name="s")

    @functools.partial(
        pl.kernel,
        mesh=mesh,
        out_type=[
            jax.ShapeDtypeStruct((T,), jnp.int32),
            jax.ShapeDtypeStruct((E,), jnp.int32),
        ],
        scratch_types=[
            pltpu.VMEM((E, T), jnp.float32),
            pltpu.VMEM((T,), jnp.int32),
            pltpu.VMEM((E,), jnp.int32),
        ],
    )
    def k(lg_hbm, sel_hbm, prev_hbm, lg_v, sel_v, prev_v):
        cid = lax.axis_index("c")
        sid = lax.axis_index("s")

        @pl.when(jnp.logical_and(cid == 0, sid == 0))
        def _():
            pltpu.sync_copy(lg_hbm, lg_v)
            zeros_i = jnp.zeros((16,), jnp.int32)
            neg = jnp.full((16,), -3.0e38, jnp.float32)
            iota = lax.iota(jnp.int32, 16)

            def body(e, carry):
                bv = list(carry[0:4])
                bi = list(carry[4:8])
                ev = jnp.full((16,), e, jnp.int32)
                for c in range(4):
                    v = lg_v[e, pl.ds(c * 16, 16)]
                    m = v > bv[c]
                    bv[c] = jnp.where(m, v, bv[c])
                    bi[c] = jnp.where(m, ev, bi[c])
                return tuple(bv) + tuple(bi)

            init = (neg, neg, neg, neg, zeros_i, zeros_i, zeros_i, zeros_i)
            res = lax.fori_loop(0, E, body, init)
            bi = res[4:8]
            for c in range(4):
                sel_v[pl.ds(c * 16, 16)] = bi[c]

            one = jnp.full((16,), 1, jnp.int32)
            zero = zeros_i
            lo = zero
            hi = zero
            for c in range(4):
                sh = jnp.bitwise_and(bi[c], 31)
                bit = jnp.left_shift(one, sh)
                lo = jnp.bitwise_or(lo, jnp.where(bi[c] < 32, bit, zero))
                hi = jnp.bitwise_or(hi, jnp.where(bi[c] >= 32, bit, zero))
            for off in (1, 2, 4, 8):
                ridx = jnp.bitwise_and(iota + off, 15)
                lo = jnp.bitwise_or(lo, _lane_gather(lo, ridx))
                hi = jnp.bitwise_or(hi, _lane_gather(hi, ridx))

            def fold(v, op):
                for off in (1, 2, 4, 8):
                    v = op(v, _lane_gather(v, jnp.bitwise_and(iota + off, 15)))
                return v

            def prefix_max(v):
                neg1 = jnp.full((16,), -1, jnp.int32)
                for off in (1, 2, 4, 8):
                    g = _lane_gather(v, jnp.maximum(iota - off, 0))
                    v = jnp.maximum(v, jnp.where(iota >= off, g, neg1))
                return v

            ev_big = jnp.full((16,), E, jnp.int32)
            f = ev_big
            us = []
            for ec in range(4):
                word = lo if ec < 2 else hi
                sh = iota + (16 if (ec % 2) else 0)
                u = jnp.bitwise_and(lax.shift_right_logical(word, sh), one)
                us.append(u)
                ids = iota + ec * 16
                f = jnp.minimum(f, fold(jnp.where(u == 1, ids, ev_big),
                                        jnp.minimum))

            carry = jnp.full((16,), -1, jnp.int32)
            for ec in range(4):
                ids = iota + ec * 16
                pc = prefix_max(jnp.where(us[ec] == 1, ids, -1))
                pc = jnp.maximum(pc, carry)
                carry = fold(pc, jnp.maximum)
                pc = jnp.where(pc < 0, f, pc)
                prev_v[pl.ds(ec * 16, 16)] = pc

            pltpu.sync_copy(sel_v, sel_hbm)
            pltpu.sync_copy(prev_v, prev_hbm)

    return k(logits_t)


HD1 = DF // 2
HD2 = DM // 2


def _moe_body(prev_sm, x_ref, b1_ref, b2_ref, bs1_ref, bs2_ref, selv_ref,
              w1_hbm, w2_hbm, ws1_hbm, ws2_hbm, out_ref,
              a10, a11, a20, a21, b10, b11, b20, b21,
              sA0, sA1, sA2, sA3, sB0, sB1, sB2, sB3, perm_sm):
    def pbody(e, cnt):
        @pl.when(prev_sm[e] == e)
        def _():
            perm_sm[cnt] = e
        return cnt + jnp.where(prev_sm[e] == e, 1, 0)

    nu = lax.fori_loop(0, E, pbody, jnp.int32(0))

    def start_fetch(q, d10, d11, d20, d21, s0, s1, s2, s3):
        @pl.when(q == 0)
        def _():
            pltpu.make_async_copy(ws1_hbm.at[pl.ds(0, HD1), :], d10, s0).start()
            pltpu.make_async_copy(ws1_hbm.at[pl.ds(HD1, HD1), :], d11, s1).start()
            pltpu.make_async_copy(ws2_hbm.at[pl.ds(0, HD2), :], d20, s2).start()
            pltpu.make_async_copy(ws2_hbm.at[pl.ds(HD2, HD2), :], d21, s3).start()

        @pl.when(q > 0)
        def _():
            e = perm_sm[jnp.maximum(q - 1, 0)]
            pltpu.make_async_copy(w1_hbm.at[e, pl.ds(0, HD1), :], d10, s0).start()
            pltpu.make_async_copy(w1_hbm.at[e, pl.ds(HD1, HD1), :], d11, s1).start()
            pltpu.make_async_copy(w2_hbm.at[e, pl.ds(0, HD2), :], d20, s2).start()
            pltpu.make_async_copy(w2_hbm.at[e, pl.ds(HD2, HD2), :], d21, s3).start()

    def wait_fetch(d10, d11, d20, d21, s0, s1, s2, s3):
        pltpu.make_async_copy(w1_hbm.at[0, pl.ds(0, HD1), :], d10, s0).wait()
        pltpu.make_async_copy(w1_hbm.at[0, pl.ds(0, HD1), :], d11, s1).wait()
        pltpu.make_async_copy(w2_hbm.at[0, pl.ds(0, HD2), :], d20, s2).wait()
        pltpu.make_async_copy(w2_hbm.at[0, pl.ds(0, HD2), :], d21, s3).wait()

    def compute(q, d10, d11, d20, d21):
        e = perm_sm[jnp.maximum(q - 1, 0)]
        b1v = jnp.where(q == 0, bs1_ref[...], b1_ref[pl.ds(e, 1), :])
        b2v = jnp.where(q == 0, bs2_ref[...], b2_ref[pl.ds(e, 1), :])
        xv = x_ref[...]
        ha = _gelu(lax.dot_general(xv, d10[...], (((1,), (1,)), ((), ())),
                                   preferred_element_type=jnp.float32)
                   + b1v[:, :HD1])
        hb = _gelu(lax.dot_general(xv, d11[...], (((1,), (1,)), ((), ())),
                                   preferred_element_type=jnp.float32)
                   + b1v[:, HD1:])
        def down(d2):
            return (lax.dot_general(ha, d2[:, 0:HD1], (((1,), (1,)), ((), ())),
                                    preferred_element_type=jnp.float32)
                    + lax.dot_general(hb, d2[:, HD1:DF],
                                      (((1,), (1,)), ((), ())),
                                      preferred_element_type=jnp.float32))
        o = jnp.concatenate([down(d20[...]), down(d21[...])], axis=1) + b2v

        @pl.when(q == 0)
        def _():
            out_ref[...] = o

        @pl.when(q > 0)
        def _():
            out_ref[...] += jnp.where(selv_ref[...] == e, o, 0.0)

    start_fetch(jnp.int32(0), a10, a11, a20, a21, sA0, sA1, sA2, sA3)
    start_fetch(jnp.int32(1), b10, b11, b20, b21, sB0, sB1, sB2, sB3)

    def mbody(m, _):
        qa = 2 * m
        wait_fetch(a10, a11, a20, a21, sA0, sA1, sA2, sA3)
        compute(qa, a10, a11, a20, a21)

        @pl.when(qa + 2 <= nu)
        def _():
            start_fetch(qa + 2, a10, a11, a20, a21, sA0, sA1, sA2, sA3)

        @pl.when(qa + 1 <= nu)
        def _():
            wait_fetch(b10, b11, b20, b21, sB0, sB1, sB2, sB3)
            compute(qa + 1, b10, b11, b20, b21)

            @pl.when(qa + 3 <= nu)
            def _():
                start_fetch(qa + 3, b10, b11, b20, b21, sB0, sB1, sB2, sB3)

        return 0

    lax.fori_loop(0, (nu + 2) // 2, mbody, 0)


def _moe(xf, W1, b1, W2, b2, Ws1, Ws2, bs1r, bs2r, selv, prev):
    grid_spec = pltpu.PrefetchScalarGridSpec(
        num_scalar_prefetch=1,
        grid=(1,),
        in_specs=[
            pl.BlockSpec((T, DM), lambda i, p: (0, 0)),
            pl.BlockSpec((E, DF), lambda i, p: (0, 0)),
            pl.BlockSpec((E, DM), lambda i, p: (0, 0)),
            pl.BlockSpec((1, DF), lambda i, p: (0, 0)),
            pl.BlockSpec((1, DM), lambda i, p: (0, 0)),
            pl.BlockSpec((T, 1), lambda i, p: (0, 0)),
            pl.BlockSpec(memory_space=pl.ANY),
            pl.BlockSpec(memory_space=pl.ANY),
            pl.BlockSpec(memory_space=pl.ANY),
            pl.BlockSpec(memory_space=pl.ANY),
        ],
        out_specs=pl.BlockSpec((T, DM), lambda i, p: (0, 0)),
        scratch_shapes=(
            [pltpu.VMEM((HD1, DM), jnp.float32)] * 2
            + [pltpu.VMEM((HD2, DF), jnp.float32)] * 2
            + [pltpu.VMEM((HD1, DM), jnp.float32)] * 2
            + [pltpu.VMEM((HD2, DF), jnp.float32)] * 2
            + [pltpu.SemaphoreType.DMA] * 8
            + [pltpu.SMEM((E,), jnp.int32)]
        ),
    )
    return pl.pallas_call(
        _moe_body,
        grid_spec=grid_spec,
        out_shape=jax.ShapeDtypeStruct((T, DM), jnp.float32),
        compiler_params=pltpu.CompilerParams(
            dimension_semantics=("arbitrary",)),
    )(prev, xf, b1, b2, bs1r, bs2r, selv, W1, W2, Ws1, Ws2)


def kernel(x, gate_w, W1, b1, W2, b2, Ws1, bs1, Ws2, bs2):
    bsz, seq, dm = x.shape
    xf = x.reshape(-1, dm)

    logits_t = _router_logits(xf, gate_w)
    sel, prev = _dispatch(logits_t)
    selv = sel.reshape(T, 1)

    out = _moe(xf, W1, b1, W2, b2, Ws1, Ws2,
               bs1.reshape(1, DF), bs2.reshape(1, DM), selv, prev)
    return out.reshape(bsz, seq, dm)
